# Initial kernel scaffold; baseline (speedup 1.0000x reference)
import jax
import jax.numpy as jnp
from jax.experimental import pallas as pl


def kernel(x, edge_index, batch, W0, b0, gn_w0, gn_b0, gn_s0, W1, b1, gn_w1, gn_b1, gn_s1, W2, b2, gn_w2, gn_b2, gn_s2):
    raise NotImplementedError("write your pallas kernel here")


# speedup vs baseline: 7.8101x; 7.8101x over previous
import functools

import jax
import jax.numpy as jnp
from jax import lax
from jax.experimental import pallas as pl
from jax.experimental.pallas import tpu as pltpu
from jax.experimental.pallas import tpu_sc as plsc

N = 10000
E = 320000
C = 128
G = 16
EPS = 1e-5

NC = 2
NS = 16
NW = NC * NS
K = 128
EPAD = ((E + NW * K - 1) // (NW * K)) * (NW * K)
EPT = EPAD // NW
NCHUNK = EPT // K
ACC_N = 10240
RPT = ACC_N // NS


@functools.cache
def _mesh():
  return plsc.VectorSubcoreMesh(core_axis_name="c", subcore_axis_name="s",
                                num_cores=NC, num_subcores=NS)


def _sc_agg_body(hs_hbm, src_hbm, dst_hbm, zeros_hbm, out_hbm,
                 acc_sh, src_v, dst_v, rows_v, sem):
  c = lax.axis_index("c")
  s = lax.axis_index("s")
  wid = c * NS + s
  pltpu.sync_copy(zeros_hbm.at[pl.ds(s * RPT, RPT)],
                  acc_sh.at[pl.ds(s * RPT, RPT)])
  plsc.subcore_barrier()
  base = wid * EPT

  def chunk(g, carry):
    off = base + g * K
    pltpu.sync_copy(src_hbm.at[pl.ds(off, K)], src_v)
    pltpu.sync_copy(dst_hbm.at[pl.ds(off, K)], dst_v)
    pltpu.async_copy(hs_hbm.at[src_v], rows_v, sem).wait()
    pltpu.sync_copy(rows_v, acc_sh.at[dst_v], add=True)
    return carry

  lax.fori_loop(0, NCHUNK, chunk, 0)
  plsc.subcore_barrier()
  pltpu.sync_copy(acc_sh.at[pl.ds(s * RPT, RPT)],
                  out_hbm.at[c, pl.ds(s * RPT, RPT)])


@functools.cache
def _sc_agg_kernel():
  return pl.kernel(
      _sc_agg_body,
      out_type=jax.ShapeDtypeStruct((NC, ACC_N, C), jnp.float32),
      mesh=_mesh(),
      scratch_types=[
          pltpu.VMEM_SHARED((ACC_N, C), jnp.float32),
          pltpu.VMEM((K,), jnp.int32),
          pltpu.VMEM((K,), jnp.int32),
          pltpu.VMEM((K, C), jnp.float32),
          pltpu.SemaphoreType.DMA,
      ],
  )


def _sc_agg(hs, src_p, dst_p, zeros_acc):
  return _sc_agg_kernel()(hs, src_p, dst_p, zeros_acc)


def _sc_deg_body(dst_hbm, ones_hbm, zeros_hbm, out_hbm,
                 acc_sh, dst_v, ones_v):
  c = lax.axis_index("c")
  s = lax.axis_index("s")
  wid = c * NS + s
  pltpu.sync_copy(zeros_hbm.at[pl.ds(s * RPT, RPT)],
                  acc_sh.at[pl.ds(s * RPT, RPT)])
  pltpu.sync_copy(ones_hbm, ones_v)
  plsc.subcore_barrier()
  base = wid * EPT

  def chunk(g, carry):
    off = base + g * K
    pltpu.sync_copy(dst_hbm.at[pl.ds(off, K)], dst_v)
    pltpu.sync_copy(ones_v, acc_sh.at[dst_v], add=True)
    return carry

  lax.fori_loop(0, NCHUNK, chunk, 0)
  plsc.subcore_barrier()
  pltpu.sync_copy(acc_sh.at[pl.ds(s * RPT, RPT)],
                  out_hbm.at[pl.ds(c * ACC_N + s * RPT, RPT)])


@functools.cache
def _sc_deg_kernel():
  return pl.kernel(
      _sc_deg_body,
      out_type=jax.ShapeDtypeStruct((NC * ACC_N,), jnp.float32),
      mesh=_mesh(),
      scratch_types=[
          pltpu.VMEM_SHARED((ACC_N,), jnp.float32),
          pltpu.VMEM((K,), jnp.int32),
          pltpu.VMEM((K,), jnp.float32),
      ],
  )


def _sc_deg(dst_p, ones_k, zeros_n):
  return _sc_deg_kernel()(dst_p, ones_k, zeros_n).reshape(NC, ACC_N)


def _tc_pre_body(degacc_ref, x_ref, w_ref, hs_ref, dinv_ref):
  deg = degacc_ref[0, :] + degacc_ref[1, :] + 1.0
  dinv = lax.rsqrt(deg)[:N][:, None]
  dinv_ref[...] = dinv
  hs_ref[...] = jnp.dot(x_ref[...], w_ref[...],
                        preferred_element_type=jnp.float32) * dinv


def _tc_pre(degacc, x, w):
  return pl.pallas_call(
      _tc_pre_body,
      out_shape=(jax.ShapeDtypeStruct((N, C), jnp.float32),
                 jax.ShapeDtypeStruct((N, 1), jnp.float32)),
  )(degacc, x, w)


BLK = 1000
NBLK = N // BLK

_GIOTA = None


def _layer_impl(acc_ref, hs_ref, dinv_ref, batch_ref, b_ref,
                gnw_ref, gnb_ref, gns_ref, h_ref, hsn_ref, pre_ref,
                wn_ref=None):
  giota = lax.broadcasted_iota(jnp.int32, (G, 1), 0)

  def pass1(i, carry):
    sums, cnts = carry
    sl = pl.ds(i * BLK, BLK)
    pre = dinv_ref[sl, :] * (acc_ref[0, sl, :] + acc_ref[1, sl, :]
                             + hs_ref[sl, :]) + b_ref[...][None, :]
    pre_ref[sl, :] = pre
    bt = batch_ref[sl, :]
    for g in range(G):
      m = bt == g
      sel = (giota == g).astype(jnp.float32)
      sums = sums + sel * jnp.sum(jnp.where(m, pre, 0.0), axis=0,
                                  keepdims=True)
      cnts = cnts + sel * jnp.sum(m.astype(jnp.float32))
    return sums, cnts

  sums, cnts = lax.fori_loop(
      0, NBLK, pass1,
      (jnp.zeros((G, C), jnp.float32), jnp.zeros((G, 1), jnp.float32)))
  cnts = jnp.maximum(cnts, 1.0)
  mean = sums / cnts

  def pass2(i, sq_sums):
    sl = pl.ds(i * BLK, BLK)
    bt = batch_ref[sl, :]
    msel = jnp.zeros((BLK, C), jnp.float32)
    for g in range(G):
      msel = jnp.where(bt == g, mean[g][None, :], msel)
    out2 = pre_ref[sl, :] - msel * gns_ref[...][None, :]
    pre_ref[sl, :] = out2
    for g in range(G):
      sel = (giota == g).astype(jnp.float32)
      sq_sums = sq_sums + sel * jnp.sum(
          jnp.where(bt == g, out2 * out2, 0.0), axis=0, keepdims=True)
    return sq_sums

  sq_sums = lax.fori_loop(0, NBLK, pass2, jnp.zeros((G, C), jnp.float32))
  inv_std = lax.rsqrt(sq_sums / cnts + EPS)

  def pass3(i, carry):
    sl = pl.ds(i * BLK, BLK)
    bt = batch_ref[sl, :]
    ssel = jnp.zeros((BLK, C), jnp.float32)
    for g in range(G):
      ssel = jnp.where(bt == g, inv_std[g][None, :], ssel)
    out2 = pre_ref[sl, :]
    hn = gnw_ref[...][None, :] * out2 * ssel + gnb_ref[...][None, :]
    h = jnp.where(hn >= 0, hn, 0.01 * hn)
    h_ref[sl, :] = h
    if wn_ref is not None:
      hsn_ref[sl, :] = jnp.dot(h, wn_ref[...],
                               preferred_element_type=jnp.float32) \
          * dinv_ref[sl, :]
    return carry

  lax.fori_loop(0, NBLK, pass3, 0)


def _tc_layer_body(acc_ref, hs_ref, dinv_ref, batch_ref, b_ref,
                   gnw_ref, gnb_ref, gns_ref, wn_ref, h_ref, hsn_ref,
                   pre_ref):
  _layer_impl(acc_ref, hs_ref, dinv_ref, batch_ref, b_ref, gnw_ref,
              gnb_ref, gns_ref, h_ref, hsn_ref, pre_ref, wn_ref=wn_ref)


def _tc_layer(acc, hs, dinv, batch2d, b, gn_w, gn_b, gn_s, w_next):
  return pl.pallas_call(
      _tc_layer_body,
      out_shape=(jax.ShapeDtypeStruct((N, C), jnp.float32),
                 jax.ShapeDtypeStruct((N, C), jnp.float32)),
      scratch_shapes=[pltpu.VMEM((N, C), jnp.float32)],
  )(acc, hs, dinv, batch2d, b, gn_w, gn_b, gn_s, w_next)


def _tc_last_body(acc_ref, hs_ref, dinv_ref, batch_ref, b_ref,
                  gnw_ref, gnb_ref, gns_ref, h_ref, pre_ref):
  _layer_impl(acc_ref, hs_ref, dinv_ref, batch_ref, b_ref, gnw_ref,
              gnb_ref, gns_ref, h_ref, None, pre_ref, wn_ref=None)


def _tc_last(acc, hs, dinv, batch2d, b, gn_w, gn_b, gn_s):
  return pl.pallas_call(
      _tc_last_body,
      out_shape=jax.ShapeDtypeStruct((N, C), jnp.float32),
      scratch_shapes=[pltpu.VMEM((N, C), jnp.float32)],
  )(acc, hs, dinv, batch2d, b, gn_w, gn_b, gn_s)


def kernel(x, edge_index, batch, W0, b0, gn_w0, gn_b0, gn_s0,
           W1, b1, gn_w1, gn_b1, gn_s1, W2, b2, gn_w2, gn_b2, gn_s2):
  src = edge_index[0].astype(jnp.int32)
  dst = edge_index[1].astype(jnp.int32)
  pad = EPAD - E
  src_p = jnp.concatenate([src, jnp.zeros((pad,), jnp.int32)])
  dst_p = jnp.concatenate([dst, jnp.full((pad,), N, jnp.int32)])
  zeros_acc = jnp.zeros((ACC_N, C), jnp.float32)
  zeros_n = jnp.zeros((ACC_N,), jnp.float32)
  ones_k = jnp.ones((K,), jnp.float32)
  batch2d = batch.astype(jnp.int32)[:, None]

  degacc = _sc_deg(dst_p, ones_k, zeros_n)
  hs, dinv = _tc_pre(degacc, x, W0)

  acc = _sc_agg(hs, src_p, dst_p, zeros_acc)
  h0, hs = _tc_layer(acc, hs, dinv, batch2d, b0, gn_w0, gn_b0, gn_s0, W1)

  acc = _sc_agg(hs, src_p, dst_p, zeros_acc)
  h1, hs = _tc_layer(acc, hs, dinv, batch2d, b1, gn_w1, gn_b1, gn_s1, W2)

  acc = _sc_agg(hs, src_p, dst_p, zeros_acc)
  h2 = _tc_last(acc, hs, dinv, batch2d, b2, gn_w2, gn_b2, gn_s2)

  return (h2, (h0, h1, h2))
